# baseline (speedup 1.0000x reference)
import functools

import jax
import jax.numpy as jnp
from jax import lax
from jax.experimental import pallas as pl
from jax.experimental.pallas import tpu as pltpu
from jax.experimental.pallas import tpu_sc as plsc

_MAXLEN = 200
_D = 64
_B = 4096
_NC, _NS = 2, 16
_NW = _NC * _NS
_G = _B // _NW
_NG = _MAXLEN
_NBI = 2
_NBO = 3
_NCHUNK = 4
_XELEMS = _G * _NG // _NCHUNK


def _body(tok_hbm, x_hbm, pos_hbm, out_hbm, xstage_v, idx_v, pos_v,
          bi0, bi1, bo0, bo1, bo2, g0, g1, o0, o1, o2):
    bins = [bi0, bi1]
    bouts = [bo0, bo1, bo2]
    gsems = [g0, g1]
    osems = [o0, o1, o2]

    wid = lax.axis_index("s") * _NC + lax.axis_index("c")
    base = wid * _G
    pltpu.sync_copy(pos_hbm, pos_v)

    rows16 = [lax.iota(jnp.int32, 16) + 16 * t for t in range(2)]
    bpc = _G // _NCHUNK
    for c4 in range(_NCHUNK):
        pltpu.sync_copy(x_hbm.at[pl.ds(base + bpc * c4, bpc)], xstage_v)

        @plsc.parallel_loop(0, _NG, step=1, unroll=4)
        def _transpose(j, c4=c4):
            col = jnp.full((16,), j, jnp.int32)
            for t in range(2):
                idx_v[j, pl.ds(bpc * c4 + 16 * t, 16)] = plsc.load_gather(
                    xstage_v, [rows16[t], col])

    def fire_gather(j, b):
        pltpu.async_copy(tok_hbm.at[idx_v.at[j]], bins[b], gsems[b])

    def wait_gather(j, b):
        pltpu.make_async_copy(tok_hbm.at[idx_v.at[j]], bins[b], gsems[b]).wait()

    def fire_write(j, b):
        pltpu.async_copy(bouts[b], out_hbm.at[pl.ds(base, _G), j], osems[b])

    def wait_write(j, b):
        pltpu.make_async_copy(
            bouts[b], out_hbm.at[pl.ds(base, _G), j], osems[b]).wait()

    def compute(j, bi, bo):
        lane0 = (j & 1) * _D
        pvs = [pos_v[j >> 1, pl.ds(lane0 + c * 16, 16)]
               for c in range(_D // 16)]

        @plsc.parallel_loop(0, _G, step=1, unroll=8)
        def add_row(i):
            for c in range(_D // 16):
                sl = pl.ds(c * 16, 16)
                bouts[bo][i, sl] = bins[bi][i, sl] + pvs[c]

    fire_gather(0, 0)
    fire_gather(1, 1)

    for j in range(3):
        wait_gather(j, j % _NBI)
        compute(j, j % _NBI, j % _NBO)
        fire_write(j, j % _NBO)
        fire_gather(j + _NBI, j % _NBI)

    def round_body(r, carry):
        j0 = 3 + r * 6
        for p in range(6):
            j = j0 + p
            bi = (3 + p) % _NBI
            bo = (3 + p) % _NBO
            wait_gather(j, bi)
            wait_write(j - _NBO, bo)
            compute(j, bi, bo)
            fire_write(j, bo)
            fire_gather(j + _NBI, bi)
        return carry

    lax.fori_loop(0, 32, round_body, 0)

    for j in range(195, 200):
        bi = j % _NBI
        bo = j % _NBO
        wait_gather(j, bi)
        wait_write(j - _NBO, bo)
        compute(j, bi, bo)
        fire_write(j, bo)
        if j + _NBI < _NG:
            fire_gather(j + _NBI, bi)

    for j in range(197, 200):
        wait_write(j, j % _NBO)


_emb = functools.partial(
    pl.kernel,
    out_type=jax.ShapeDtypeStruct((_B, _MAXLEN, _D), jnp.float32),
    mesh=plsc.VectorSubcoreMesh(
        core_axis_name="c", subcore_axis_name="s",
        num_cores=_NC, num_subcores=_NS),
    scratch_types=(
        [pltpu.VMEM((_G // _NCHUNK, _NG), jnp.int32),
         pltpu.VMEM((_NG, _G), jnp.int32),
         pltpu.VMEM((_MAXLEN // 2, 128), jnp.float32)]
        + [pltpu.VMEM((_G, 128), jnp.float32) for _ in range(_NBI)]
        + [pltpu.VMEM((_G, _D), jnp.float32) for _ in range(_NBO)]
        + [pltpu.SemaphoreType.DMA for _ in range(_NBI + _NBO)]
    ),
    compiler_params=pltpu.CompilerParams(
        use_tc_tiling_on_sc=True, needs_layout_passes=False),
)(_body)


def kernel(x, token_table, pos_table):
    tok_pad = jnp.pad(token_table, ((0, 0), (0, 128 - _D)))
    pos2 = jnp.reshape(pos_table, (_MAXLEN // 2, 128))
    return _emb(tok_pad, x, pos2)


# speedup vs baseline: 1.6208x; 1.6208x over previous
import functools

import jax
import jax.numpy as jnp
from jax import lax
from jax.experimental import pallas as pl
from jax.experimental.pallas import tpu as pltpu
from jax.experimental.pallas import tpu_sc as plsc

_MAXLEN = 200
_D = 64
_B = 4096
_NC, _NS = 2, 16
_NW = _NC * _NS
_G = _B // _NW
_NG = _MAXLEN
_NBUF = 4
_NROUND = _NG // _NBUF


def _body(tok_hbm, x_hbm, pos_hbm, out_hbm, xrow_v, idx_v, pos_v,
          bi0, bi1, bi2, bi3, bo0, bo1, bo2, bo3,
          g0, g1, g2, g3, o0, o1, o2, o3):
    bins = [bi0, bi1, bi2, bi3]
    bouts = [bo0, bo1, bo2, bo3]
    gsems = [g0, g1, g2, g3]
    osems = [o0, o1, o2, o3]

    wid = lax.axis_index("s") * _NC + lax.axis_index("c")
    base = wid * _G
    pltpu.sync_copy(x_hbm.at[pl.ds(base, _G)], xrow_v)
    pltpu.sync_copy(pos_hbm, pos_v)

    rows16 = [lax.iota(jnp.int32, 16) + 16 * k for k in range(_G // 16)]

    @plsc.parallel_loop(0, _NG, step=1, unroll=4)
    def _transpose(j):
        col = jnp.full((16,), j, jnp.int32)
        for k in range(_G // 16):
            idx_v[j, pl.ds(16 * k, 16)] = plsc.load_gather(
                xrow_v, [rows16[k], col])

    def fire_gather(j, b):
        pltpu.async_copy(tok_hbm.at[idx_v.at[j]], bins[b], gsems[b])

    def wait_gather(j, b):
        pltpu.make_async_copy(tok_hbm.at[idx_v.at[j]], bins[b], gsems[b]).wait()

    def fire_write(j, b):
        pltpu.async_copy(
            bouts[b], out_hbm.at[pl.ds(base, _G), j, pl.ds(0, _D)], osems[b])

    def wait_write(j, b):
        pltpu.make_async_copy(
            bouts[b], out_hbm.at[pl.ds(base, _G), j, pl.ds(0, _D)],
            osems[b]).wait()

    def compute(j, b):
        pvs = [pos_v[j, pl.ds(c * 16, 16)] for c in range(_D // 16)]

        @plsc.parallel_loop(0, _G, step=1, unroll=8)
        def add_row(i):
            for c in range(_D // 16):
                sl = pl.ds(c * 16, 16)
                bouts[b][i, sl] = bins[b][i, sl] + pvs[c]

    for b in range(_NBUF):
        fire_gather(b, b)

    for b in range(_NBUF):
        wait_gather(b, b)
        compute(b, b)
        fire_write(b, b)
        fire_gather(_NBUF + b, b)

    def round_body(r, carry):
        j0 = r * _NBUF
        for b in range(_NBUF):
            j = j0 + b
            wait_gather(j, b)
            wait_write(j - _NBUF, b)
            compute(j, b)
            fire_write(j, b)
            fire_gather(j + _NBUF, b)
        return carry

    lax.fori_loop(1, _NROUND - 1, round_body, 0)

    j0 = (_NROUND - 1) * _NBUF
    for b in range(_NBUF):
        j = j0 + b
        wait_gather(j, b)
        wait_write(j - _NBUF, b)
        compute(j, b)
        fire_write(j, b)

    for b in range(_NBUF):
        wait_write(j0 + b, b)


_emb = functools.partial(
    pl.kernel,
    out_type=jax.ShapeDtypeStruct((_B, _MAXLEN, 128), jnp.float32),
    mesh=plsc.VectorSubcoreMesh(
        core_axis_name="c", subcore_axis_name="s",
        num_cores=_NC, num_subcores=_NS),
    scratch_types=(
        [pltpu.VMEM((_G, _NG), jnp.int32),
         pltpu.VMEM((_NG, _G), jnp.int32),
         pltpu.VMEM((_MAXLEN, _D), jnp.float32)]
        + [pltpu.VMEM((_G, _D), jnp.float32) for _ in range(2 * _NBUF)]
        + [pltpu.SemaphoreType.DMA for _ in range(2 * _NBUF)]
    ),
    compiler_params=pltpu.CompilerParams(
        use_tc_tiling_on_sc=False, needs_layout_passes=False),
)(_body)


def kernel(x, token_table, pos_table):
    return _emb(token_table, x, pos_table)[..., :_D]


# speedup vs baseline: 1.6236x; 1.0018x over previous
import functools

import jax
import jax.numpy as jnp
from jax import lax
from jax.experimental import pallas as pl
from jax.experimental.pallas import tpu as pltpu
from jax.experimental.pallas import tpu_sc as plsc

_MAXLEN = 200
_D = 64
_B = 4096
_NC, _NS = 2, 16
_NW = _NC * _NS
_G = _B // _NW
_NG = _MAXLEN
_NBUF = 8


def _body(tok_hbm, x_hbm, pos_hbm, out_hbm, xrow_v, idx_v, pos_v,
          b0, b1, b2, b3, b4, b5, b6, b7,
          g0, g1, g2, g3, g4, g5, g6, g7, o0, o1, o2, o3, o4, o5, o6, o7):
    bufs = [b0, b1, b2, b3, b4, b5, b6, b7]
    gsems = [g0, g1, g2, g3, g4, g5, g6, g7]
    osems = [o0, o1, o2, o3, o4, o5, o6, o7]

    wid = lax.axis_index("s") * _NC + lax.axis_index("c")
    base = wid * _G
    pltpu.sync_copy(x_hbm.at[pl.ds(base, _G)], xrow_v)
    pltpu.sync_copy(pos_hbm, pos_v)

    rows16 = [lax.iota(jnp.int32, 16) + 16 * k for k in range(_G // 16)]

    @plsc.parallel_loop(0, _NG, step=1, unroll=4)
    def _transpose(j):
        col = jnp.full((16,), j, jnp.int32)
        for k in range(_G // 16):
            idx_v[j, pl.ds(16 * k, 16)] = plsc.load_gather(
                xrow_v, [rows16[k], col])

    def fire_gather(j, b):
        pltpu.async_copy(tok_hbm.at[idx_v.at[j]], bufs[b], gsems[b])

    def wait_gather(j, b):
        pltpu.make_async_copy(tok_hbm.at[idx_v.at[j]], bufs[b], gsems[b]).wait()

    def fire_write(j, b):
        pltpu.async_copy(
            bufs[b], out_hbm.at[pl.ds(base, _G), j, pl.ds(0, _D)], osems[b])

    def wait_write(j, b):
        pltpu.make_async_copy(
            bufs[b], out_hbm.at[pl.ds(base, _G), j, pl.ds(0, _D)],
            osems[b]).wait()

    def compute(j, b):
        pvs = [pos_v[j, pl.ds(c * 16, 16)] for c in range(_D // 16)]

        @plsc.parallel_loop(0, _G, step=1, unroll=8)
        def add_row(i):
            for c in range(_D // 16):
                plsc.addupdate(bufs[b].at[i, pl.ds(c * 16, 16)], pvs[c])

    for j in range(4):
        fire_gather(j, j)

    for j in range(4):
        wait_gather(j, j)
        compute(j, j)
        fire_write(j, j)
        fire_gather(j + 4, j + 4)

    def round_body(r, carry):
        j0 = 4 + r * _NBUF
        for p in range(_NBUF):
            j = j0 + p
            b = (4 + p) % _NBUF
            bn = p % _NBUF
            wait_gather(j, b)
            compute(j, b)
            fire_write(j, b)
            wait_write(j - 4, bn)
            fire_gather(j + 4, bn)
        return carry

    lax.fori_loop(0, (_NG - _NBUF) // _NBUF, round_body, 0)

    for j in range(196, 200):
        b = j % _NBUF
        wait_gather(j, b)
        compute(j, b)
        fire_write(j, b)
        wait_write(j - 4, (j - 4) % _NBUF)

    for j in range(196, 200):
        wait_write(j, j % _NBUF)


_emb = functools.partial(
    pl.kernel,
    out_type=jax.ShapeDtypeStruct((_B, _MAXLEN, 128), jnp.float32),
    mesh=plsc.VectorSubcoreMesh(
        core_axis_name="c", subcore_axis_name="s",
        num_cores=_NC, num_subcores=_NS),
    scratch_types=(
        [pltpu.VMEM((_G, _NG), jnp.int32),
         pltpu.VMEM((_NG, _G), jnp.int32),
         pltpu.VMEM((_MAXLEN, _D), jnp.float32)]
        + [pltpu.VMEM((_G, _D), jnp.float32) for _ in range(_NBUF)]
        + [pltpu.SemaphoreType.DMA for _ in range(2 * _NBUF)]
    ),
    compiler_params=pltpu.CompilerParams(
        use_tc_tiling_on_sc=False, needs_layout_passes=False),
)(_body)


def kernel(x, token_table, pos_table):
    return _emb(token_table, x, pos_table)[..., :_D]
